# baseline (speedup 1.0000x reference)
import jax
import jax.numpy as jnp
from jax import lax
from jax.experimental import pallas as pl
from jax.experimental.pallas import tpu as pltpu
from jax.experimental.pallas import tpu_sc as plsc

_NC = 2
_NS = 16
_NW = _NC * _NS
_L = 16
_BC = 128
_DT = 8
_DR = 8
_TB = 128


def _tpose_body(tt_hbm, x_hbm, tin, tout, isem, osem):
    wid = lax.axis_index("s") * _NC + lax.axis_index("c")
    dim, vocab = tt_hbm.shape
    nblk = (vocab + _TB - 1) // _TB
    per = (nblk + _NW - 1) // _NW
    base = wid * per
    iota = lax.iota(jnp.int32, _L)

    def col0(b):
        return lax.min(b * _TB, vocab - _TB)

    pltpu.async_copy(tt_hbm.at[:, pl.ds(col0(base), _TB)], tin.at[0], isem)

    def body(j, carry):
        b = base + j
        s = lax.rem(j, 2)
        ns = lax.rem(j + 1, 2)

        @pl.when(b < nblk)
        def _():
            pltpu.make_async_copy(
                tt_hbm.at[:, pl.ds(col0(b), _TB)], tin.at[s], isem
            ).wait()

            @pl.when((j + 1 < per) & (b + 1 < nblk))
            def _():
                pltpu.async_copy(
                    tt_hbm.at[:, pl.ds(col0(b + 1), _TB)], tin.at[ns], isem
                )

            @pl.when(j >= 1)
            def _():
                pltpu.make_async_copy(
                    tout.at[ns, :, pl.ds(0, dim)],
                    x_hbm.at[pl.ds(0, _TB)],
                    osem,
                ).wait()

            for d in range(dim):
                dsplat = iota * 0 + d
                loaded = [
                    tin[s, d, pl.ds(cg * _L, _L)] for cg in range(_TB // _L)
                ]
                for cg in range(_TB // _L):
                    plsc.store_scatter(
                        tout.at[s], [iota + cg * _L, dsplat], loaded[cg]
                    )

            pltpu.async_copy(
                tout.at[s, :, pl.ds(0, dim)],
                x_hbm.at[pl.ds(col0(b), _TB)],
                osem,
            )

        return carry

    lax.fori_loop(0, per, body, 0)
    pltpu.make_async_copy(
        tout.at[0, :, pl.ds(0, dim)], x_hbm.at[pl.ds(0, _TB)], osem
    ).wait()


def _embed_body(idx_hbm, table_hbm, out_hbm, idx_v, gbuf, wbuf, gsem, ssem):
    wid = lax.axis_index("s") * _NC + lax.axis_index("c")
    n = idx_v.shape[0]
    dim = table_hbm.shape[1]
    nbt = out_hbm.shape[2]

    pltpu.sync_copy(idx_hbm.at[wid], idx_v)
    pltpu.async_copy(table_hbm.at[idx_v.at[0]], gbuf.at[0], gsem)
    pltpu.async_copy(table_hbm.at[idx_v.at[1]], gbuf.at[1], gsem)

    iota = lax.iota(jnp.int32, _L)

    def body(j, carry):
        s = lax.rem(j, 3)
        ws = lax.rem(j, 2)
        wns = lax.rem(j + 1, 2)
        blk = wid * n + j
        h = blk // nbt
        bt = lax.rem(blk, nbt)

        pltpu.make_async_copy(table_hbm.at[idx_v.at[j]], gbuf.at[s], gsem).wait()

        @pl.when(j + 2 < n)
        def _():
            pltpu.async_copy(
                table_hbm.at[idx_v.at[j + 2]], gbuf.at[lax.rem(j + 2, 3)], gsem
            )

        @pl.when(j >= 1)
        def _():
            pltpu.make_async_copy(
                wbuf.at[wns, :, :, pl.ds(0, _BC)], out_hbm.at[0, :, 0], ssem
            ).wait()

        for bc in range(_BC):
            bsplat = iota * 0 + bc
            loaded = [
                gbuf[s, bc, pl.ds(g * _L, _L)] for g in range(dim // _L)
            ]
            for g in range(dim // _L):
                dt_idx = (iota // _DR) + (2 * g)
                dr_idx = iota % _DR
                plsc.store_scatter(
                    wbuf.at[ws], [dt_idx, dr_idx, bsplat], loaded[g]
                )

        pltpu.async_copy(
            wbuf.at[ws, :, :, pl.ds(0, _BC)], out_hbm.at[h, :, bt], ssem
        )
        return carry

    lax.fori_loop(0, n, body, 0)
    pltpu.make_async_copy(
        wbuf.at[0, :, :, pl.ds(0, _BC)], out_hbm.at[0, :, 0], ssem
    ).wait()


def kernel(x, table):
    b0, b1 = x.shape
    vocab, dim = table.shape
    batch = b0 * b1
    nbt = b0 // _BC
    nblocks = b1 * nbt
    n = nblocks // _NW

    idxb = x.astype(jnp.int32).T.reshape(b1, nbt, _BC).reshape(nblocks, _BC)
    idxb = idxb.reshape(_NW, n, _BC)

    mesh = plsc.VectorSubcoreMesh(core_axis_name="c", subcore_axis_name="s")
    params = pltpu.CompilerParams(
        use_tc_tiling_on_sc=False, needs_layout_passes=False
    )

    tpose = pl.kernel(
        _tpose_body,
        out_type=jax.ShapeDtypeStruct((vocab, dim), table.dtype),
        mesh=mesh,
        scratch_types=[
            pltpu.VMEM((2, dim, _TB), jnp.float32),
            pltpu.VMEM((2, _TB, dim + 1), jnp.float32),
            pltpu.SemaphoreType.DMA,
            pltpu.SemaphoreType.DMA,
        ],
        compiler_params=params,
    )
    table_lin = tpose(table.T)

    run = pl.kernel(
        _embed_body,
        out_type=jax.ShapeDtypeStruct((b1, _DT, nbt, _DR, _BC), table.dtype),
        mesh=mesh,
        scratch_types=[
            pltpu.VMEM((n, _BC), jnp.int32),
            pltpu.VMEM((3, _BC, dim), jnp.float32),
            pltpu.VMEM((2, _DT, _DR, _BC + 1), jnp.float32),
            pltpu.SemaphoreType.DMA,
            pltpu.SemaphoreType.DMA,
        ],
        compiler_params=params,
    )
    out5 = run(idxb, table_lin)
    return out5.transpose(2, 4, 0, 1, 3).reshape(b0, b1, dim)


# speedup vs baseline: 6.7171x; 6.7171x over previous
import jax
import jax.numpy as jnp
from jax import lax
from jax.experimental import pallas as pl
from jax.experimental.pallas import tpu as pltpu
from jax.experimental.pallas import tpu_sc as plsc

_NC = 2
_NS = 16
_NW = _NC * _NS
_L = 16
_BC = 128
_DT = 8
_DR = 8


def _embed_body(idx_hbm, table_hbm, out_hbm, idx_v, gbuf, wbuf, gsem, ssem):
    wid = lax.axis_index("s") * _NC + lax.axis_index("c")
    n = idx_v.shape[0]
    dim = table_hbm.shape[1]
    nbt = out_hbm.shape[2]

    pltpu.sync_copy(idx_hbm.at[wid], idx_v)
    pltpu.async_copy(table_hbm.at[idx_v.at[0]], gbuf.at[0], gsem)
    pltpu.async_copy(table_hbm.at[idx_v.at[1]], gbuf.at[1], gsem)
    pltpu.async_copy(table_hbm.at[idx_v.at[2]], gbuf.at[2], gsem)

    iota = lax.iota(jnp.int32, _L)

    def body(j, carry):
        s = lax.rem(j, 4)
        ws = lax.rem(j, 4)
        wns = lax.rem(j + 1, 4)
        blk = wid * n + j
        h = blk // nbt
        bt = lax.rem(blk, nbt)

        pltpu.make_async_copy(table_hbm.at[idx_v.at[j]], gbuf.at[s], gsem).wait()

        @pl.when(j + 3 < n)
        def _():
            pltpu.async_copy(
                table_hbm.at[idx_v.at[j + 3]], gbuf.at[lax.rem(j + 3, 4)], gsem
            )

        @pl.when(j >= 3)
        def _():
            pltpu.make_async_copy(
                wbuf.at[wns, :, :, pl.ds(0, _BC)], out_hbm.at[0, :, 0], ssem
            ).wait()

        ng = dim // _L
        for bc0 in range(0, _BC, 2):
            loaded = [
                gbuf[s, bc0 + (k // ng), pl.ds((k % ng) * _L, _L)]
                for k in range(2 * ng)
            ]
            for k in range(2 * ng):
                g = k % ng
                bsplat = iota * 0 + (bc0 + k // ng)
                dt_idx = (iota // _DR) + (2 * g)
                dr_idx = iota % _DR
                plsc.store_scatter(
                    wbuf.at[ws], [dt_idx, dr_idx, bsplat], loaded[k]
                )

        pltpu.async_copy(
            wbuf.at[ws, :, :, pl.ds(0, _BC)], out_hbm.at[h, :, bt], ssem
        )
        return carry

    lax.fori_loop(0, n, body, 0)
    pltpu.make_async_copy(
        wbuf.at[pl.ds(0, 3), :, :, pl.ds(0, _BC)],
        out_hbm.at[pl.ds(0, 3), :, 0],
        ssem,
    ).wait()


def kernel(x, table):
    b0, b1 = x.shape
    vocab, dim = table.shape
    batch = b0 * b1
    nbt = b0 // _BC
    nblocks = b1 * nbt
    n = nblocks // _NW

    idxb = x.astype(jnp.int32).T.reshape(b1, nbt, _BC).reshape(nblocks, _BC)
    idxb = idxb.reshape(_NW, n, _BC)

    mesh = plsc.VectorSubcoreMesh(core_axis_name="c", subcore_axis_name="s")
    params = pltpu.CompilerParams(
        use_tc_tiling_on_sc=False, needs_layout_passes=False
    )

    run = pl.kernel(
        _embed_body,
        out_type=jax.ShapeDtypeStruct((b1, _DT, nbt, _DR, _BC), table.dtype),
        mesh=mesh,
        scratch_types=[
            pltpu.VMEM((n, _BC), jnp.int32),
            pltpu.VMEM((4, _BC, dim), jnp.float32),
            pltpu.VMEM((4, _DT, _DR, _BC + 1), jnp.float32),
            pltpu.SemaphoreType.DMA,
            pltpu.SemaphoreType.DMA,
        ],
        compiler_params=params,
    )
    out5 = run(idxb, table)
    return out5.transpose(2, 4, 0, 1, 3).reshape(b0, b1, dim)
